# Initial kernel scaffold; baseline (speedup 1.0000x reference)
import jax
import jax.numpy as jnp
from jax.experimental import pallas as pl


def kernel(user, item, user_table, item_table, user_bias, item_bias):
    raise NotImplementedError("write your pallas kernel here")


# speedup vs baseline: 51.9520x; 51.9520x over previous
import dataclasses
import functools

import jax
import jax.numpy as jnp
from jax import lax
from jax.experimental import pallas as pl
from jax.experimental.pallas import tpu as pltpu
from jax.experimental.pallas import tpu_sc as plsc

B = 4096
L = 200
D = 128
NC = 2
NS = 16
NW = NC * NS
BL = B // NW
S0, S1 = 104, 96
NCHUNK = D // 16


def _take(vec, idx):
    dnums = lax.GatherDimensionNumbers(
        offset_dims=(), collapsed_slice_dims=(0,), start_index_map=(0,))
    return lax.gather(vec, idx[:, None], dnums, slice_sizes=(1,),
                      mode=lax.GatherScatterMode.PROMISE_IN_BOUNDS)


def _lane_sum_bcast(acc):
    iota = lax.iota(jnp.int32, 16)
    for sh in (1, 2, 4, 8):
        acc = acc + _take(acc, lax.bitwise_xor(iota, sh))
    return acc


def _mf_sc(user, item_flat, user_table, item_table, user_bias, item_bias):
    mesh = plsc.VectorSubcoreMesh(core_axis_name="c", subcore_axis_name="s")
    cp = pltpu.CompilerParams()
    if "needs_layout_passes" in pltpu.CompilerParams.__dataclass_fields__:
        cp = dataclasses.replace(cp, needs_layout_passes=False)

    @functools.partial(
        pl.kernel,
        out_type=jax.ShapeDtypeStruct((B, L), jnp.float32),
        mesh=mesh,
        compiler_params=cp,
        scratch_types=[
            pltpu.VMEM((BL,), jnp.int32),
            pltpu.VMEM((BL * L,), jnp.int32),
            pltpu.VMEM((BL, D), jnp.float32),
            pltpu.VMEM((BL,), jnp.float32),
            pltpu.VMEM((L, D), jnp.float32),
            pltpu.VMEM((L, D), jnp.float32),
            pltpu.VMEM((L,), jnp.float32),
            pltpu.VMEM((L,), jnp.float32),
            pltpu.VMEM((BL, L), jnp.float32),
            pltpu.SemaphoreType.DMA,
            pltpu.SemaphoreType.DMA,
            pltpu.SemaphoreType.DMA,
        ],
    )
    def k(user_hbm, item_hbm, utab_hbm, itab_hbm, ubias_hbm, ibias_hbm,
          out_hbm, uidx_v, idx_v, urows_v, ub_v, rows_a, rows_b, ib_a, ib_b,
          out_v, sem_a, sem_b, sem0):
        wid = lax.axis_index("s") * NC + lax.axis_index("c")
        base = wid * BL

        pltpu.sync_copy(user_hbm.at[pl.ds(base, BL)], uidx_v)
        pltpu.sync_copy(item_hbm.at[pl.ds(base * L, BL * L)], idx_v)
        pltpu.async_copy(utab_hbm.at[uidx_v], urows_v, sem0).wait()
        pltpu.async_copy(ubias_hbm.at[uidx_v], ub_v, sem0).wait()

        def idx_views(b):
            o = pl.multiple_of(b * L, 8)
            return (idx_v.at[pl.ds(o, S0)],
                    idx_v.at[pl.ds(pl.multiple_of(b * L + S0, 8), S1)])

        def fire(b, rows, ib, sem):
            i1, i2 = idx_views(b)
            pltpu.async_copy(itab_hbm.at[i1], rows.at[pl.ds(0, S0)], sem)
            pltpu.async_copy(itab_hbm.at[i2], rows.at[pl.ds(S0, S1)], sem)
            pltpu.async_copy(ibias_hbm.at[i1], ib.at[pl.ds(0, S0)], sem)
            pltpu.async_copy(ibias_hbm.at[i2], ib.at[pl.ds(S0, S1)], sem)

        def drain(b, rows, ib, sem):
            i1, i2 = idx_views(b)
            pltpu.make_async_copy(itab_hbm.at[i1], rows.at[pl.ds(0, S0)],
                                  sem).wait()
            pltpu.make_async_copy(itab_hbm.at[i2], rows.at[pl.ds(S0, S1)],
                                  sem).wait()
            pltpu.make_async_copy(ibias_hbm.at[i1], ib.at[pl.ds(0, S0)],
                                  sem).wait()
            pltpu.make_async_copy(ibias_hbm.at[i2], ib.at[pl.ds(S0, S1)],
                                  sem).wait()

        lane_iota = lax.iota(jnp.int32, 16)

        def compute(b, rows, ib):
            u = [urows_v[b, pl.ds(16 * c, 16)] for c in range(NCHUNK)]
            ub_chunk = ub_v[pl.ds((b // 16) * 16, 16)]
            ubs = _take(ub_chunk, jnp.full((16,), lax.rem(b, 16), jnp.int32))

            @pl.loop(0, 13)
            def _(g):
                off = jnp.minimum(16 * g, L - 16)
                out16 = jnp.zeros((16,), jnp.float32)
                for j in range(16):
                    row = off + j
                    acc = rows[row, pl.ds(0, 16)] * u[0]
                    for c in range(1, NCHUNK):
                        acc = acc + rows[row, pl.ds(16 * c, 16)] * u[c]
                    acc = _lane_sum_bcast(acc)
                    out16 = jnp.where(lane_iota == j, acc, out16)
                out16 = out16 + ib[pl.ds(off, 16)] + ubs
                out_v[b, pl.ds(off, 16)] = out16

        fire(0, rows_a, ib_a, sem_a)

        @pl.loop(0, BL, step=2)
        def _(b):
            fire(b + 1, rows_b, ib_b, sem_b)
            drain(b, rows_a, ib_a, sem_a)
            compute(b, rows_a, ib_a)

            @pl.when(b + 2 < BL)
            def _():
                fire(b + 2, rows_a, ib_a, sem_a)

            drain(b + 1, rows_b, ib_b, sem_b)
            compute(b + 1, rows_b, ib_b)

        pltpu.sync_copy(out_v, out_hbm.at[pl.ds(base, BL)])

    return k(user, item_flat, user_table, item_table, user_bias, item_bias)


def kernel(user, item, user_table, item_table, user_bias, item_bias):
    item_flat = item.reshape(B * L).astype(jnp.int32)
    user = user.astype(jnp.int32)
    return _mf_sc(user, item_flat, user_table, item_table, user_bias,
                  item_bias)
